# baseline (speedup 1.0000x reference)
import functools

import jax
import jax.numpy as jnp
from jax import lax
from jax.experimental import pallas as pl
from jax.experimental.pallas import tpu as pltpu
from jax.experimental.pallas import tpu_sc as plsc

VOCAB = 1000000
DIM = 64
PDIM = 2 * DIM
B = 4096
L = 50
TOTAL = B * L

_info = plsc.get_sparse_core_info()
NC = _info.num_cores
NS = _info.num_subcores
NW = NC * NS
PER_W = TOTAL // NW
BATCH_W = B // NW
CB = 4
CHUNK = CB * L
NCHUNK = PER_W // CHUNK
GRP = 8

_mesh = plsc.VectorSubcoreMesh(core_axis_name="c", subcore_axis_name="s")


@functools.partial(
    pl.kernel,
    mesh=_mesh,
    out_type=jax.ShapeDtypeStruct((B, L, DIM), jnp.float32),
    scratch_types=[
        pltpu.VMEM((PER_W + 16,), jnp.int32),
        pltpu.VMEM((2, CHUNK, PDIM), jnp.float32),
        pltpu.VMEM((2, CHUNK, DIM), jnp.float32),
        pltpu.SemaphoreType.DMA,
        pltpu.SemaphoreType.DMA,
    ],
)
def _gather(table_hbm, idx_hbm, out_hbm, idx_v, rows_v, pack_v, gsem, osem):
    wid = lax.axis_index("s") * NC + lax.axis_index("c")
    base_b = wid * BATCH_W
    pltpu.sync_copy(idx_hbm.at[wid], idx_v.at[pl.ds(0, PER_W)])

    def fire_chunk(i, bsel):
        def grp_body(k, carry):
            rvec = idx_v[pl.ds(i * CHUNK + k * GRP, 16)]
            for j in range(GRP):
                pltpu.async_copy(
                    table_hbm.at[pl.ds(rvec[j] >> 1, 1)],
                    rows_v.at[bsel, pl.ds(k * GRP + j, 1)],
                    gsem)
            return carry

        lax.fori_loop(0, CHUNK // GRP, grp_body, 0)

    def drain_gather(bsel):
        pltpu.make_async_copy(
            table_hbm.at[pl.ds(0, CHUNK)], rows_v.at[bsel], gsem).wait()

    def select_chunk(i, bsel):
        def grp_body(k, carry):
            rvec = idx_v[pl.ds(i * CHUNK + k * GRP, 16)]
            for j in range(GRP):
                row = k * GRP + j
                half = (rvec[j] & 1) * DIM
                for c in range(DIM // 16):
                    v = rows_v[bsel, row, pl.ds(half + c * 16, 16)]
                    pack_v[bsel, row, pl.ds(c * 16, 16)] = v
            return carry

        lax.fori_loop(0, CHUNK // GRP, grp_body, 0)

    def out_copies(i, bsel):
        return [
            pltpu.make_async_copy(
                pack_v.at[bsel, pl.ds(q * L, L)],
                out_hbm.at[base_b + i * CB + q],
                osem)
            for q in range(CB)
        ]

    fire_chunk(0, 0)

    def chunk_body(i, carry):
        bsel = i % 2
        nsel = 1 - bsel
        drain_gather(bsel)

        @pl.when(i + 1 < NCHUNK)
        def _():
            fire_chunk(i + 1, nsel)

        select_chunk(i, bsel)

        @pl.when(i >= 1)
        def _():
            for cp in out_copies(i - 1, nsel):
                cp.wait()

        for cp in out_copies(i, bsel):
            cp.start()
        return carry

    lax.fori_loop(0, NCHUNK, chunk_body, 0)
    for cp in out_copies(NCHUNK - 1, (NCHUNK - 1) % 2):
        cp.wait()


def kernel(inputs, table):
    idx = inputs.reshape(NW, PER_W).astype(jnp.int32)
    return _gather(table.reshape(VOCAB // 2, PDIM), idx)


# speedup vs baseline: 1.5910x; 1.5910x over previous
import functools

import jax
import jax.numpy as jnp
from jax import lax
from jax.experimental import pallas as pl
from jax.experimental.pallas import tpu as pltpu
from jax.experimental.pallas import tpu_sc as plsc

VOCAB = 1000000
DIM = 64
B = 4096
L = 50
TOTAL = B * L

_info = plsc.get_sparse_core_info()
NC = _info.num_cores
NS = _info.num_subcores
NW = NC * NS
PER_W = TOTAL // NW
BATCH_W = B // NW
CB = 8
CHUNK = CB * L
NCHUNK = PER_W // CHUNK

_mesh = plsc.VectorSubcoreMesh(core_axis_name="c", subcore_axis_name="s")


@functools.partial(
    pl.kernel,
    mesh=_mesh,
    out_type=jax.ShapeDtypeStruct((B, L, DIM), jnp.float32),
    scratch_types=[
        pltpu.VMEM((PER_W,), jnp.int32),
        pltpu.VMEM((2, CHUNK, DIM), jnp.float32),
        pltpu.SemaphoreType.DMA,
        pltpu.SemaphoreType.DMA,
        pltpu.SemaphoreType.DMA,
    ],
)
def _gather(table_hbm, idx_hbm, out_hbm, idx_v, rows_v, gsem, osem0, osem1):
    wid = lax.axis_index("s") * NC + lax.axis_index("c")
    base_b = wid * BATCH_W
    osems = (osem0, osem1)
    pltpu.sync_copy(idx_hbm.at[wid], idx_v)

    def fire_chunk(i, bsel):
        def grp_body(k, carry):
            rvec = idx_v[pl.ds(i * CHUNK + k * 16, 16)]
            for l in range(16):
                pltpu.async_copy(
                    table_hbm.at[pl.ds(rvec[l], 1)],
                    rows_v.at[bsel, pl.ds(k * 16 + l, 1)],
                    gsem)
            return carry

        lax.fori_loop(0, CHUNK // 16, grp_body, 0)

    def drain_chunk(bsel):
        pltpu.make_async_copy(
            table_hbm.at[pl.ds(0, CHUNK)], rows_v.at[bsel], gsem).wait()

    pend_o = [None, None]
    fire_chunk(0, 0)
    for i in range(NCHUNK):
        b = i % 2
        nb = (i + 1) % 2
        drain_chunk(b)
        if i + 1 < NCHUNK:
            if pend_o[nb] is not None:
                for cp in pend_o[nb]:
                    cp.wait()
            fire_chunk(i + 1, nb)
        pend_o[b] = [
            pltpu.async_copy(
                rows_v.at[b, pl.ds(q * L, L)],
                out_hbm.at[base_b + i * CB + q],
                osems[b])
            for q in range(CB)
        ]
    for lst in pend_o:
        for cp in lst:
            cp.wait()


def kernel(inputs, table):
    idx = inputs.reshape(NW, PER_W).astype(jnp.int32)
    return _gather(table, idx)
